# baseline (speedup 1.0000x reference)
import jax
import jax.numpy as jnp
from jax import lax
from jax.experimental import pallas as pl
from jax.experimental.pallas import tpu as pltpu
from jax.experimental.pallas import tpu_sc as plsc

N = 10000
E = 320000
DIN = 128
DH = 64
DC = 40
DCP = 48
DEGW = 8

NC, NS = 2, 16
NW = NC * NS
CH = 128
NB = 4
NITER = 80
EPAD = NW * CH * NITER
NP = 10240
RPT = NP // NS
NG = NITER // NB - 1

_SC_MESH = plsc.VectorSubcoreMesh(core_axis_name="c", subcore_axis_name="s")
_SC_PARAMS = pltpu.CompilerParams(use_tc_tiling_on_sc=False)


def _sc_agg_body(esrc3, edst3, table, zeros, out,
                 src_all, dst_all, rows, acc, sem_g, sem_s):
    cid = lax.axis_index("c")
    sid = lax.axis_index("s")
    wid = cid * NS + sid
    pltpu.sync_copy(zeros.at[pl.ds(sid * RPT, RPT)],
                    acc.at[pl.ds(sid * RPT, RPT)])
    pltpu.sync_copy(esrc3.at[wid], src_all)
    pltpu.sync_copy(edst3.at[wid], dst_all)
    plsc.subcore_barrier()

    def fire_g(b, i):
        pltpu.async_copy(table.at[src_all.at[i]], rows[b], sem_g[b])

    def wait_g(b):
        pltpu.make_async_copy(table.at[src_all.at[0]], rows[b], sem_g[b]).wait()

    def fire_s(b, i):
        pltpu.async_copy(rows[b], acc.at[dst_all.at[i]], sem_s[b], add=True)

    def wait_s(b):
        pltpu.make_async_copy(rows[b], acc.at[dst_all.at[0]], sem_s[b]).wait()

    for b in range(NB):
        fire_g(b, b)

    @pl.loop(0, NG)
    def _(g):
        i0 = NB * g
        for b in range(NB):
            wait_g(b)
            fire_s(b, i0 + b)
        for b in range(NB):
            wait_s(b)
            fire_g(b, i0 + NB + b)

    for b in range(NB):
        wait_g(b)
        fire_s(b, NB * NG + b)
    for b in range(NB):
        wait_s(b)

    plsc.subcore_barrier()
    pltpu.sync_copy(acc.at[pl.ds(sid * RPT, RPT)],
                    out.at[cid, pl.ds(sid * RPT, RPT)])


def _make_sc_agg(D):
    return pl.kernel(
        _sc_agg_body,
        out_type=jax.ShapeDtypeStruct((NC, NP, D), jnp.float32),
        mesh=_SC_MESH,
        compiler_params=_SC_PARAMS,
        scratch_types=[
            pltpu.VMEM((NITER, CH), jnp.int32),
            pltpu.VMEM((NITER, CH), jnp.int32),
            [pltpu.VMEM((CH, D), jnp.float32)] * NB,
            pltpu.VMEM_SHARED((NP, D), jnp.float32),
            [pltpu.SemaphoreType.DMA] * NB,
            [pltpu.SemaphoreType.DMA] * NB,
        ],
        name=f"sc_gcn_agg_{D}",
    )


def _sc_deg_body(edst3, ones, zeros, out, dst_all, ones_v, acc, sem):
    cid = lax.axis_index("c")
    sid = lax.axis_index("s")
    wid = cid * NS + sid
    pltpu.sync_copy(zeros.at[pl.ds(sid * RPT, RPT)],
                    acc.at[pl.ds(sid * RPT, RPT)])
    pltpu.sync_copy(edst3.at[wid], dst_all)
    pltpu.sync_copy(ones, ones_v)
    plsc.subcore_barrier()

    @pl.loop(0, NITER)
    def _(i):
        pltpu.async_copy(ones_v, acc.at[dst_all.at[i]], sem, add=True)

    @pl.loop(0, NITER)
    def _(i):
        pltpu.make_async_copy(ones_v, acc.at[dst_all.at[0]], sem).wait()

    plsc.subcore_barrier()
    pltpu.sync_copy(acc.at[pl.ds(sid * RPT, RPT)],
                    out.at[cid, pl.ds(sid * RPT, RPT)])


_sc_deg = pl.kernel(
    _sc_deg_body,
    out_type=jax.ShapeDtypeStruct((NC, NP, DEGW), jnp.float32),
    mesh=_SC_MESH,
    compiler_params=_SC_PARAMS,
    scratch_types=[
        pltpu.VMEM((NITER, CH), jnp.int32),
        pltpu.VMEM((CH, DEGW), jnp.float32),
        pltpu.VMEM_SHARED((NP, DEGW), jnp.float32),
        pltpu.SemaphoreType.DMA,
    ],
    name="sc_gcn_deg",
)

_sc_agg64 = _make_sc_agg(DH)
_sc_agg48 = _make_sc_agg(DCP)


RB = 1000
GRID = N // RB


def _tc1_body(x_ref, w1_ref, degp_ref, dis_ref, xws1_ref):
    deg = degp_ref[0, :, 0:1] + degp_ref[1, :, 0:1] + 1.0
    dis = 1.0 / jnp.sqrt(deg)
    xw = jnp.dot(x_ref[...], w1_ref[...], preferred_element_type=jnp.float32)
    dis_ref[...] = dis
    xws1_ref[...] = xw * dis


def _tc2_body(xws1_ref, p_ref, dis_ref, b1_ref, w2_ref, xws2_ref):
    dis = dis_ref[...]
    h = dis * (p_ref[0] + p_ref[1] + xws1_ref[...]) + b1_ref[...]
    h = jnp.maximum(h, 0.0)
    xws2_ref[...] = jnp.dot(
        h, w2_ref[...], preferred_element_type=jnp.float32) * dis


def _tc3_body(xws2_ref, p_ref, dis_ref, b2_ref, out_ref):
    o = dis_ref[...] * (p_ref[0] + p_ref[1] + xws2_ref[...]) + b2_ref[...]
    mask = lax.broadcasted_iota(jnp.int32, (1, DCP), 1) < DC
    m = jnp.max(jnp.where(mask, o, -jnp.inf), axis=1, keepdims=True)
    s = jnp.sum(jnp.where(mask, jnp.exp(o - m), 0.0), axis=1, keepdims=True)
    out_ref[...] = (o - m - jnp.log(s))[:, :DC]


def _row_spec(d):
    return pl.BlockSpec((RB, d), lambda i: (i, 0))


def _full_spec(shape):
    nd = len(shape)
    return pl.BlockSpec(shape, lambda i: (0,) * nd)


_tc1 = pl.pallas_call(
    _tc1_body,
    grid=(GRID,),
    in_specs=[_row_spec(DIN), _full_spec((DIN, DH)),
              pl.BlockSpec((NC, RB, DEGW), lambda i: (0, i, 0))],
    out_specs=[_row_spec(1), _row_spec(DH)],
    out_shape=[jax.ShapeDtypeStruct((N, 1), jnp.float32),
               jax.ShapeDtypeStruct((N, DH), jnp.float32)],
)

_tc2 = pl.pallas_call(
    _tc2_body,
    grid=(GRID,),
    in_specs=[_row_spec(DH), pl.BlockSpec((NC, RB, DH), lambda i: (0, i, 0)),
              _row_spec(1), _full_spec((1, DH)), _full_spec((DH, DCP))],
    out_specs=_row_spec(DCP),
    out_shape=jax.ShapeDtypeStruct((N, DCP), jnp.float32),
)

_tc3 = pl.pallas_call(
    _tc3_body,
    grid=(GRID,),
    in_specs=[_row_spec(DCP), pl.BlockSpec((NC, RB, DCP), lambda i: (0, i, 0)),
              _row_spec(1), _full_spec((1, DCP))],
    out_specs=pl.BlockSpec((RB, DC), lambda i: (i, 0)),
    out_shape=jax.ShapeDtypeStruct((N, DC), jnp.float32),
)


def kernel(x, edge_index, W1, b1, W2, b2):
    ei = edge_index.astype(jnp.int32)
    esrc3 = jnp.pad(ei[0], (0, EPAD - E)).reshape(NW, NITER, CH)
    edst3 = jnp.pad(ei[1], (0, EPAD - E),
                    constant_values=NP - 1).reshape(NW, NITER, CH)
    zeros64 = jnp.zeros((NP, DH), jnp.float32)
    zeros48 = jnp.zeros((NP, DCP), jnp.float32)
    zeros_d = jnp.zeros((NP, DEGW), jnp.float32)
    ones_d = jnp.ones((CH, DEGW), jnp.float32)
    w2p = jnp.pad(W2, ((0, 0), (0, DCP - DC)))
    b2p = jnp.pad(b2, (0, DCP - DC)).reshape(1, DCP)
    b1r = b1.reshape(1, DH)

    degp = _sc_deg(edst3, ones_d, zeros_d)
    dis, xws1 = _tc1(x, W1, degp)
    p1 = _sc_agg64(esrc3, edst3, xws1, zeros64)
    xws2 = _tc2(xws1, p1, dis, b1r, w2p)
    p2 = _sc_agg48(esrc3, edst3, xws2, zeros48)
    return _tc3(xws2, p2, dis, b2p)


# speedup vs baseline: 1.0210x; 1.0210x over previous
import jax
import jax.numpy as jnp
from jax import lax
from jax.experimental import pallas as pl
from jax.experimental.pallas import tpu as pltpu
from jax.experimental.pallas import tpu_sc as plsc

N = 10000
E = 320000
DIN = 128
DH = 64
DC = 40
DCP = 48
DEGW = 8

NC, NS = 2, 16
NW = NC * NS
CH = 128
NB = 4
NITER = 80
EPAD = NW * CH * NITER
NP = 10240
RPT = NP // NS
LAG = 2

_SC_MESH = plsc.VectorSubcoreMesh(core_axis_name="c", subcore_axis_name="s")
_SC_PARAMS = pltpu.CompilerParams(use_tc_tiling_on_sc=False)


def _sc_agg_body(esrc3, edst3, table, zeros, out,
                 src_all, dst_all, rows, acc, sem_g, sem_s):
    cid = lax.axis_index("c")
    sid = lax.axis_index("s")
    wid = cid * NS + sid
    pltpu.sync_copy(zeros.at[pl.ds(sid * RPT, RPT)],
                    acc.at[pl.ds(sid * RPT, RPT)])
    pltpu.sync_copy(esrc3.at[wid], src_all)
    pltpu.sync_copy(edst3.at[wid], dst_all)
    plsc.subcore_barrier()

    def fire_g(b, i):
        pltpu.async_copy(table.at[src_all.at[i]], rows[b], sem_g[b])

    def wait_g(b):
        pltpu.make_async_copy(table.at[src_all.at[0]], rows[b], sem_g[b]).wait()

    def fire_s(b, i):
        pltpu.async_copy(rows[b], acc.at[dst_all.at[i]], sem_s[b], add=True)

    def wait_s(b):
        pltpu.make_async_copy(rows[b], acc.at[dst_all.at[0]], sem_s[b]).wait()

    @pl.loop(0, NITER // NB)
    def _(g):
        for b in range(NB):
            i = NB * g + b

            @pl.when(g > 0)
            def _():
                wait_s(b)
            fire_g(b, i)
            bw = (b - LAG) % NB

            @pl.when(i >= LAG)
            def _():
                wait_g(bw)
                fire_s(bw, i - LAG)

    for t in range(LAG):
        bw = (NITER - LAG + t) % NB
        wait_g(bw)
        fire_s(bw, NITER - LAG + t)
    for b in range(NB):
        wait_s(b)

    plsc.subcore_barrier()
    pltpu.sync_copy(acc.at[pl.ds(sid * RPT, RPT)],
                    out.at[cid, pl.ds(sid * RPT, RPT)])


def _make_sc_agg(D):
    return pl.kernel(
        _sc_agg_body,
        out_type=jax.ShapeDtypeStruct((NC, NP, D), jnp.float32),
        mesh=_SC_MESH,
        compiler_params=_SC_PARAMS,
        scratch_types=[
            pltpu.VMEM((NITER, CH), jnp.int32),
            pltpu.VMEM((NITER, CH), jnp.int32),
            [pltpu.VMEM((CH, D), jnp.float32)] * NB,
            pltpu.VMEM_SHARED((NP, D), jnp.float32),
            [pltpu.SemaphoreType.DMA] * NB,
            [pltpu.SemaphoreType.DMA] * NB,
        ],
        name=f"sc_gcn_agg_{D}",
    )


def _sc_deg_body(edst3, ones, zeros, out, dst_all, ones_v, acc, sem):
    cid = lax.axis_index("c")
    sid = lax.axis_index("s")
    wid = cid * NS + sid
    pltpu.sync_copy(zeros.at[pl.ds(sid * RPT, RPT)],
                    acc.at[pl.ds(sid * RPT, RPT)])
    pltpu.sync_copy(edst3.at[wid], dst_all)
    pltpu.sync_copy(ones, ones_v)
    plsc.subcore_barrier()

    @pl.loop(0, NITER)
    def _(i):
        pltpu.async_copy(ones_v, acc.at[dst_all.at[i]], sem, add=True)

    @pl.loop(0, NITER)
    def _(i):
        pltpu.make_async_copy(ones_v, acc.at[dst_all.at[0]], sem).wait()

    plsc.subcore_barrier()
    pltpu.sync_copy(acc.at[pl.ds(sid * RPT, RPT)],
                    out.at[cid, pl.ds(sid * RPT, RPT)])


_sc_deg = pl.kernel(
    _sc_deg_body,
    out_type=jax.ShapeDtypeStruct((NC, NP, DEGW), jnp.float32),
    mesh=_SC_MESH,
    compiler_params=_SC_PARAMS,
    scratch_types=[
        pltpu.VMEM((NITER, CH), jnp.int32),
        pltpu.VMEM((CH, DEGW), jnp.float32),
        pltpu.VMEM_SHARED((NP, DEGW), jnp.float32),
        pltpu.SemaphoreType.DMA,
    ],
    name="sc_gcn_deg",
)

_sc_agg64 = _make_sc_agg(DH)
_sc_agg48 = _make_sc_agg(DCP)


RB = 1000
GRID = N // RB


def _tc1_body(x_ref, w1_ref, degp_ref, dis_ref, xws1_ref):
    deg = degp_ref[0, :, 0:1] + degp_ref[1, :, 0:1] + 1.0
    dis = 1.0 / jnp.sqrt(deg)
    xw = jnp.dot(x_ref[...], w1_ref[...], preferred_element_type=jnp.float32)
    dis_ref[...] = dis
    xws1_ref[...] = xw * dis


def _tc2_body(xws1_ref, p_ref, dis_ref, b1_ref, w2_ref, xws2_ref):
    dis = dis_ref[...]
    h = dis * (p_ref[0] + p_ref[1] + xws1_ref[...]) + b1_ref[...]
    h = jnp.maximum(h, 0.0)
    xws2_ref[...] = jnp.dot(
        h, w2_ref[...], preferred_element_type=jnp.float32) * dis


def _tc3_body(xws2_ref, p_ref, dis_ref, b2_ref, out_ref):
    o = dis_ref[...] * (p_ref[0] + p_ref[1] + xws2_ref[...]) + b2_ref[...]
    mask = lax.broadcasted_iota(jnp.int32, (1, DCP), 1) < DC
    m = jnp.max(jnp.where(mask, o, -jnp.inf), axis=1, keepdims=True)
    s = jnp.sum(jnp.where(mask, jnp.exp(o - m), 0.0), axis=1, keepdims=True)
    out_ref[...] = (o - m - jnp.log(s))[:, :DC]


def _row_spec(d):
    return pl.BlockSpec((RB, d), lambda i: (i, 0))


def _full_spec(shape):
    nd = len(shape)
    return pl.BlockSpec(shape, lambda i: (0,) * nd)


_tc1 = pl.pallas_call(
    _tc1_body,
    grid=(GRID,),
    in_specs=[_row_spec(DIN), _full_spec((DIN, DH)),
              pl.BlockSpec((NC, RB, DEGW), lambda i: (0, i, 0))],
    out_specs=[_row_spec(1), _row_spec(DH)],
    out_shape=[jax.ShapeDtypeStruct((N, 1), jnp.float32),
               jax.ShapeDtypeStruct((N, DH), jnp.float32)],
)

_tc2 = pl.pallas_call(
    _tc2_body,
    grid=(GRID,),
    in_specs=[_row_spec(DH), pl.BlockSpec((NC, RB, DH), lambda i: (0, i, 0)),
              _row_spec(1), _full_spec((1, DH)), _full_spec((DH, DCP))],
    out_specs=_row_spec(DCP),
    out_shape=jax.ShapeDtypeStruct((N, DCP), jnp.float32),
)

_tc3 = pl.pallas_call(
    _tc3_body,
    grid=(GRID,),
    in_specs=[_row_spec(DCP), pl.BlockSpec((NC, RB, DCP), lambda i: (0, i, 0)),
              _row_spec(1), _full_spec((1, DCP))],
    out_specs=pl.BlockSpec((RB, DC), lambda i: (i, 0)),
    out_shape=jax.ShapeDtypeStruct((N, DC), jnp.float32),
)


def kernel(x, edge_index, W1, b1, W2, b2):
    ei = edge_index.astype(jnp.int32)
    esrc3 = jnp.pad(ei[0], (0, EPAD - E)).reshape(NW, NITER, CH)
    edst3 = jnp.pad(ei[1], (0, EPAD - E),
                    constant_values=NP - 1).reshape(NW, NITER, CH)
    zeros64 = jnp.zeros((NP, DH), jnp.float32)
    zeros48 = jnp.zeros((NP, DCP), jnp.float32)
    zeros_d = jnp.zeros((NP, DEGW), jnp.float32)
    ones_d = jnp.ones((CH, DEGW), jnp.float32)
    w2p = jnp.pad(W2, ((0, 0), (0, DCP - DC)))
    b2p = jnp.pad(b2, (0, DCP - DC)).reshape(1, DCP)
    b1r = b1.reshape(1, DH)

    degp = _sc_deg(edst3, ones_d, zeros_d)
    dis, xws1 = _tc1(x, W1, degp)
    p1 = _sc_agg64(esrc3, edst3, xws1, zeros64)
    xws2 = _tc2(xws1, p1, dis, b1r, w2p)
    p2 = _sc_agg48(esrc3, edst3, xws2, zeros48)
    return _tc3(xws2, p2, dis, b2p)


# speedup vs baseline: 1.0247x; 1.0036x over previous
import jax
import jax.numpy as jnp
from jax import lax
from jax.experimental import pallas as pl
from jax.experimental.pallas import tpu as pltpu
from jax.experimental.pallas import tpu_sc as plsc

N = 10000
E = 320000
DIN = 128
DH = 64
DC = 40
DCP = 48
DEGW = 8

NC, NS = 2, 16
NW = NC * NS
CH = 128
NB = 8
NITER = 80
EPAD = NW * CH * NITER
NP = 10240
RPT = NP // NS
LAG = 4

_SC_MESH = plsc.VectorSubcoreMesh(core_axis_name="c", subcore_axis_name="s")
_SC_PARAMS = pltpu.CompilerParams(use_tc_tiling_on_sc=False)


def _sc_agg_body(esrc3, edst3, table, zeros, out,
                 src_all, dst_all, rows, acc, sem_g, sem_s):
    cid = lax.axis_index("c")
    sid = lax.axis_index("s")
    wid = cid * NS + sid
    pltpu.sync_copy(zeros.at[pl.ds(sid * RPT, RPT)],
                    acc.at[pl.ds(sid * RPT, RPT)])
    pltpu.sync_copy(esrc3.at[wid], src_all)
    pltpu.sync_copy(edst3.at[wid], dst_all)
    plsc.subcore_barrier()

    def fire_g(b, i):
        pltpu.async_copy(table.at[src_all.at[i]], rows[b], sem_g[b])

    def wait_g(b):
        pltpu.make_async_copy(table.at[src_all.at[0]], rows[b], sem_g[b]).wait()

    def fire_s(b, i):
        pltpu.async_copy(rows[b], acc.at[dst_all.at[i]], sem_s[b], add=True)

    def wait_s(b):
        pltpu.make_async_copy(rows[b], acc.at[dst_all.at[0]], sem_s[b]).wait()

    @pl.loop(0, NITER // NB)
    def _(g):
        for b in range(NB):
            i = NB * g + b

            @pl.when(g > 0)
            def _():
                wait_s(b)
            fire_g(b, i)
            bw = (b - LAG) % NB

            @pl.when(i >= LAG)
            def _():
                wait_g(bw)
                fire_s(bw, i - LAG)

    for t in range(LAG):
        bw = (NITER - LAG + t) % NB
        wait_g(bw)
        fire_s(bw, NITER - LAG + t)
    for b in range(NB):
        wait_s(b)

    plsc.subcore_barrier()
    pltpu.sync_copy(acc.at[pl.ds(sid * RPT, RPT)],
                    out.at[cid, pl.ds(sid * RPT, RPT)])


def _make_sc_agg(D):
    return pl.kernel(
        _sc_agg_body,
        out_type=jax.ShapeDtypeStruct((NC, NP, D), jnp.float32),
        mesh=_SC_MESH,
        compiler_params=_SC_PARAMS,
        scratch_types=[
            pltpu.VMEM((NITER, CH), jnp.int32),
            pltpu.VMEM((NITER, CH), jnp.int32),
            [pltpu.VMEM((CH, D), jnp.float32)] * NB,
            pltpu.VMEM_SHARED((NP, D), jnp.float32),
            [pltpu.SemaphoreType.DMA] * NB,
            [pltpu.SemaphoreType.DMA] * NB,
        ],
        name=f"sc_gcn_agg_{D}",
    )


def _sc_deg_body(edst3, ones, zeros, out, dst_all, ones_v, acc, sem):
    cid = lax.axis_index("c")
    sid = lax.axis_index("s")
    wid = cid * NS + sid
    pltpu.sync_copy(zeros.at[pl.ds(sid * RPT, RPT)],
                    acc.at[pl.ds(sid * RPT, RPT)])
    pltpu.sync_copy(edst3.at[wid], dst_all)
    pltpu.sync_copy(ones, ones_v)
    plsc.subcore_barrier()

    @pl.loop(0, NITER)
    def _(i):
        pltpu.async_copy(ones_v, acc.at[dst_all.at[i]], sem, add=True)

    @pl.loop(0, NITER)
    def _(i):
        pltpu.make_async_copy(ones_v, acc.at[dst_all.at[0]], sem).wait()

    plsc.subcore_barrier()
    pltpu.sync_copy(acc.at[pl.ds(sid * RPT, RPT)],
                    out.at[cid, pl.ds(sid * RPT, RPT)])


_sc_deg = pl.kernel(
    _sc_deg_body,
    out_type=jax.ShapeDtypeStruct((NC, NP, DEGW), jnp.float32),
    mesh=_SC_MESH,
    compiler_params=_SC_PARAMS,
    scratch_types=[
        pltpu.VMEM((NITER, CH), jnp.int32),
        pltpu.VMEM((CH, DEGW), jnp.float32),
        pltpu.VMEM_SHARED((NP, DEGW), jnp.float32),
        pltpu.SemaphoreType.DMA,
    ],
    name="sc_gcn_deg",
)

_sc_agg64 = _make_sc_agg(DH)
_sc_agg48 = _make_sc_agg(DCP)


RB = 1000
GRID = N // RB


def _tc1_body(x_ref, w1_ref, degp_ref, dis_ref, xws1_ref):
    deg = degp_ref[0, :, 0:1] + degp_ref[1, :, 0:1] + 1.0
    dis = 1.0 / jnp.sqrt(deg)
    xw = jnp.dot(x_ref[...], w1_ref[...], preferred_element_type=jnp.float32)
    dis_ref[...] = dis
    xws1_ref[...] = xw * dis


def _tc2_body(xws1_ref, p_ref, dis_ref, b1_ref, w2_ref, xws2_ref):
    dis = dis_ref[...]
    h = dis * (p_ref[0] + p_ref[1] + xws1_ref[...]) + b1_ref[...]
    h = jnp.maximum(h, 0.0)
    xws2_ref[...] = jnp.dot(
        h, w2_ref[...], preferred_element_type=jnp.float32) * dis


def _tc3_body(xws2_ref, p_ref, dis_ref, b2_ref, out_ref):
    o = dis_ref[...] * (p_ref[0] + p_ref[1] + xws2_ref[...]) + b2_ref[...]
    mask = lax.broadcasted_iota(jnp.int32, (1, DCP), 1) < DC
    m = jnp.max(jnp.where(mask, o, -jnp.inf), axis=1, keepdims=True)
    s = jnp.sum(jnp.where(mask, jnp.exp(o - m), 0.0), axis=1, keepdims=True)
    out_ref[...] = (o - m - jnp.log(s))[:, :DC]


def _row_spec(d):
    return pl.BlockSpec((RB, d), lambda i: (i, 0))


def _full_spec(shape):
    nd = len(shape)
    return pl.BlockSpec(shape, lambda i: (0,) * nd)


_tc1 = pl.pallas_call(
    _tc1_body,
    grid=(GRID,),
    in_specs=[_row_spec(DIN), _full_spec((DIN, DH)),
              pl.BlockSpec((NC, RB, DEGW), lambda i: (0, i, 0))],
    out_specs=[_row_spec(1), _row_spec(DH)],
    out_shape=[jax.ShapeDtypeStruct((N, 1), jnp.float32),
               jax.ShapeDtypeStruct((N, DH), jnp.float32)],
)

_tc2 = pl.pallas_call(
    _tc2_body,
    grid=(GRID,),
    in_specs=[_row_spec(DH), pl.BlockSpec((NC, RB, DH), lambda i: (0, i, 0)),
              _row_spec(1), _full_spec((1, DH)), _full_spec((DH, DCP))],
    out_specs=_row_spec(DCP),
    out_shape=jax.ShapeDtypeStruct((N, DCP), jnp.float32),
)

_tc3 = pl.pallas_call(
    _tc3_body,
    grid=(GRID,),
    in_specs=[_row_spec(DCP), pl.BlockSpec((NC, RB, DCP), lambda i: (0, i, 0)),
              _row_spec(1), _full_spec((1, DCP))],
    out_specs=pl.BlockSpec((RB, DC), lambda i: (i, 0)),
    out_shape=jax.ShapeDtypeStruct((N, DC), jnp.float32),
)


def kernel(x, edge_index, W1, b1, W2, b2):
    ei = edge_index.astype(jnp.int32)
    esrc3 = jnp.pad(ei[0], (0, EPAD - E)).reshape(NW, NITER, CH)
    edst3 = jnp.pad(ei[1], (0, EPAD - E),
                    constant_values=NP - 1).reshape(NW, NITER, CH)
    zeros64 = jnp.zeros((NP, DH), jnp.float32)
    zeros48 = jnp.zeros((NP, DCP), jnp.float32)
    zeros_d = jnp.zeros((NP, DEGW), jnp.float32)
    ones_d = jnp.ones((CH, DEGW), jnp.float32)
    w2p = jnp.pad(W2, ((0, 0), (0, DCP - DC)))
    b2p = jnp.pad(b2, (0, DCP - DC)).reshape(1, DCP)
    b1r = b1.reshape(1, DH)

    degp = _sc_deg(edst3, ones_d, zeros_d)
    dis, xws1 = _tc1(x, W1, degp)
    p1 = _sc_agg64(esrc3, edst3, xws1, zeros64)
    xws2 = _tc2(xws1, p1, dis, b1r, w2p)
    p2 = _sc_agg48(esrc3, edst3, xws2, zeros48)
    return _tc3(xws2, p2, dis, b2p)


# speedup vs baseline: 2.0336x; 1.9846x over previous
import jax
import jax.numpy as jnp
from jax import lax
from jax.experimental import pallas as pl
from jax.experimental.pallas import tpu as pltpu
from jax.experimental.pallas import tpu_sc as plsc

N = 10000
E = 320000
DIN = 128
DH = 64
DC = 40
DCP = 48
DEGW = 8

NC, NS = 2, 16
NW = NC * NS
CH = 128
NITER = 80
EPAD = NW * CH * NITER
NP = 10240
RPT = NP // NS
HALF = NITER // 2

_SC_MESH = plsc.VectorSubcoreMesh(core_axis_name="c", subcore_axis_name="s")
_SC_PARAMS = pltpu.CompilerParams(use_tc_tiling_on_sc=False)


def _sc_agg_body(esrc3, edst3, table, zeros, out,
                 src_all, dst_all, rows_a, rows_b, tbl, acc, sem_a, sem_b):
    cid = lax.axis_index("c")
    sid = lax.axis_index("s")
    wid = cid * NS + sid
    pltpu.sync_copy(zeros.at[pl.ds(sid * RPT, RPT)],
                    acc.at[pl.ds(sid * RPT, RPT)])
    pltpu.sync_copy(table.at[pl.ds(sid * RPT, RPT)],
                    tbl.at[pl.ds(sid * RPT, RPT)])
    pltpu.sync_copy(esrc3.at[wid], src_all)
    pltpu.sync_copy(edst3.at[wid], dst_all)
    plsc.subcore_barrier()

    def fire(i, rows, sem):
        pltpu.async_copy(tbl.at[src_all.at[i]], rows, sem)

    def wait(rows, sem):
        pltpu.make_async_copy(tbl.at[src_all.at[0]], rows, sem).wait()

    def scat(i, rows):
        pltpu.sync_copy(rows, acc.at[dst_all.at[i]], add=True)

    fire(0, rows_a, sem_a)

    @pl.loop(0, HALF - 1)
    def _(j):
        i = 2 * j
        fire(i + 1, rows_b, sem_b)
        wait(rows_a, sem_a)
        scat(i, rows_a)
        fire(i + 2, rows_a, sem_a)
        wait(rows_b, sem_b)
        scat(i + 1, rows_b)

    fire(NITER - 1, rows_b, sem_b)
    wait(rows_a, sem_a)
    scat(NITER - 2, rows_a)
    wait(rows_b, sem_b)
    scat(NITER - 1, rows_b)

    plsc.subcore_barrier()
    pltpu.sync_copy(acc.at[pl.ds(sid * RPT, RPT)],
                    out.at[cid, pl.ds(sid * RPT, RPT)])


def _make_sc_agg(D):
    return pl.kernel(
        _sc_agg_body,
        out_type=jax.ShapeDtypeStruct((NC, NP, D), jnp.float32),
        mesh=_SC_MESH,
        compiler_params=_SC_PARAMS,
        scratch_types=[
            pltpu.VMEM((NITER, CH), jnp.int32),
            pltpu.VMEM((NITER, CH), jnp.int32),
            pltpu.VMEM((CH, D), jnp.float32),
            pltpu.VMEM((CH, D), jnp.float32),
            pltpu.VMEM_SHARED((NP, D), jnp.float32),
            pltpu.VMEM_SHARED((NP, D), jnp.float32),
            pltpu.SemaphoreType.DMA,
            pltpu.SemaphoreType.DMA,
        ],
        name=f"sc_gcn_agg_{D}",
    )


def _sc_deg_body(edst3, ones, zeros, out, dst_all, ones_v, acc, sem):
    cid = lax.axis_index("c")
    sid = lax.axis_index("s")
    wid = cid * NS + sid
    pltpu.sync_copy(zeros.at[pl.ds(sid * RPT, RPT)],
                    acc.at[pl.ds(sid * RPT, RPT)])
    pltpu.sync_copy(edst3.at[wid], dst_all)
    pltpu.sync_copy(ones, ones_v)
    plsc.subcore_barrier()

    @pl.loop(0, NITER)
    def _(i):
        pltpu.async_copy(ones_v, acc.at[dst_all.at[i]], sem, add=True)

    @pl.loop(0, NITER)
    def _(i):
        pltpu.make_async_copy(ones_v, acc.at[dst_all.at[0]], sem).wait()

    plsc.subcore_barrier()
    pltpu.sync_copy(acc.at[pl.ds(sid * RPT, RPT)],
                    out.at[cid, pl.ds(sid * RPT, RPT)])


_sc_deg = pl.kernel(
    _sc_deg_body,
    out_type=jax.ShapeDtypeStruct((NC, NP, DEGW), jnp.float32),
    mesh=_SC_MESH,
    compiler_params=_SC_PARAMS,
    scratch_types=[
        pltpu.VMEM((NITER, CH), jnp.int32),
        pltpu.VMEM((CH, DEGW), jnp.float32),
        pltpu.VMEM_SHARED((NP, DEGW), jnp.float32),
        pltpu.SemaphoreType.DMA,
    ],
    name="sc_gcn_deg",
)

_sc_agg64 = _make_sc_agg(DH)
_sc_agg48 = _make_sc_agg(DCP)


RB = 1024
GRID = NP // RB


def _tc1_body(x_ref, w1_ref, degp_ref, dis_ref, xws1_ref):
    deg = degp_ref[0, :, 0:1] + degp_ref[1, :, 0:1] + 1.0
    dis = 1.0 / jnp.sqrt(deg)
    xw = jnp.dot(x_ref[...], w1_ref[...], preferred_element_type=jnp.float32)
    dis_ref[...] = dis
    xws1_ref[...] = xw * dis


def _tc2_body(xws1_ref, p_ref, dis_ref, b1_ref, w2_ref, xws2_ref):
    dis = dis_ref[...]
    h = dis * (p_ref[0] + p_ref[1] + xws1_ref[...]) + b1_ref[...]
    h = jnp.maximum(h, 0.0)
    xws2_ref[...] = jnp.dot(
        h, w2_ref[...], preferred_element_type=jnp.float32) * dis


def _tc3_body(xws2_ref, p_ref, dis_ref, b2_ref, out_ref):
    o = dis_ref[...] * (p_ref[0] + p_ref[1] + xws2_ref[...]) + b2_ref[...]
    mask = lax.broadcasted_iota(jnp.int32, (1, DCP), 1) < DC
    m = jnp.max(jnp.where(mask, o, -jnp.inf), axis=1, keepdims=True)
    s = jnp.sum(jnp.where(mask, jnp.exp(o - m), 0.0), axis=1, keepdims=True)
    out_ref[...] = (o - m - jnp.log(s))[:, :DC]


def _row_spec(d):
    return pl.BlockSpec((RB, d), lambda i: (i, 0))


def _full_spec(shape):
    nd = len(shape)
    return pl.BlockSpec(shape, lambda i: (0,) * nd)


_tc1 = pl.pallas_call(
    _tc1_body,
    grid=(GRID,),
    in_specs=[_row_spec(DIN), _full_spec((DIN, DH)),
              pl.BlockSpec((NC, RB, DEGW), lambda i: (0, i, 0))],
    out_specs=[_row_spec(1), _row_spec(DH)],
    out_shape=[jax.ShapeDtypeStruct((NP, 1), jnp.float32),
               jax.ShapeDtypeStruct((NP, DH), jnp.float32)],
)

_tc2 = pl.pallas_call(
    _tc2_body,
    grid=(GRID,),
    in_specs=[_row_spec(DH), pl.BlockSpec((NC, RB, DH), lambda i: (0, i, 0)),
              _row_spec(1), _full_spec((1, DH)), _full_spec((DH, DCP))],
    out_specs=_row_spec(DCP),
    out_shape=jax.ShapeDtypeStruct((NP, DCP), jnp.float32),
)

_tc3 = pl.pallas_call(
    _tc3_body,
    grid=(GRID,),
    in_specs=[_row_spec(DCP), pl.BlockSpec((NC, RB, DCP), lambda i: (0, i, 0)),
              _row_spec(1), _full_spec((1, DCP))],
    out_specs=pl.BlockSpec((RB, DC), lambda i: (i, 0)),
    out_shape=jax.ShapeDtypeStruct((N, DC), jnp.float32),
)


def kernel(x, edge_index, W1, b1, W2, b2):
    ei = edge_index.astype(jnp.int32)
    esrc3 = jnp.pad(ei[0], (0, EPAD - E)).reshape(NW, NITER, CH)
    edst3 = jnp.pad(ei[1], (0, EPAD - E),
                    constant_values=NP - 1).reshape(NW, NITER, CH)
    zeros64 = jnp.zeros((NP, DH), jnp.float32)
    zeros48 = jnp.zeros((NP, DCP), jnp.float32)
    zeros_d = jnp.zeros((NP, DEGW), jnp.float32)
    ones_d = jnp.ones((CH, DEGW), jnp.float32)
    w2p = jnp.pad(W2, ((0, 0), (0, DCP - DC)))
    b2p = jnp.pad(b2, (0, DCP - DC)).reshape(1, DCP)
    b1r = b1.reshape(1, DH)

    degp = _sc_deg(edst3, ones_d, zeros_d)
    dis, xws1 = _tc1(x, W1, degp)
    p1 = _sc_agg64(esrc3, edst3, xws1, zeros64)
    xws2 = _tc2(xws1, p1, dis, b1r, w2p)
    p2 = _sc_agg48(esrc3, edst3, xws2, zeros48)
    return _tc3(xws2, p2, dis, b2p)


# speedup vs baseline: 2.0831x; 1.0243x over previous
import jax
import jax.numpy as jnp
from jax import lax
from jax.experimental import pallas as pl
from jax.experimental.pallas import tpu as pltpu
from jax.experimental.pallas import tpu_sc as plsc

N = 10000
E = 320000
DIN = 128
DH = 64
DC = 40
DCP = 40
DEGW = 8

NC, NS = 2, 16
NW = NC * NS
CH = 128
NITER = 80
EPAD = NW * CH * NITER
NP = 10240
RPT = NP // NS
HALF = NITER // 2

_SC_MESH = plsc.VectorSubcoreMesh(core_axis_name="c", subcore_axis_name="s")
_SC_PARAMS = pltpu.CompilerParams(use_tc_tiling_on_sc=False)


def _sc_agg_body(esrc3, edst3, table, zeros, out,
                 src_all, dst_all, rows_a, rows_b, tbl, acc, sem_a, sem_b):
    cid = lax.axis_index("c")
    sid = lax.axis_index("s")
    wid = cid * NS + sid
    pltpu.sync_copy(zeros.at[pl.ds(sid * RPT, RPT)],
                    acc.at[pl.ds(sid * RPT, RPT)])
    pltpu.sync_copy(table.at[pl.ds(sid * RPT, RPT)],
                    tbl.at[pl.ds(sid * RPT, RPT)])
    pltpu.sync_copy(esrc3.at[wid], src_all)
    pltpu.sync_copy(edst3.at[wid], dst_all)
    plsc.subcore_barrier()

    def fire(i, rows, sem):
        pltpu.async_copy(tbl.at[src_all.at[i]], rows, sem)

    def wait(rows, sem):
        pltpu.make_async_copy(tbl.at[src_all.at[0]], rows, sem).wait()

    def scat(i, rows):
        pltpu.sync_copy(rows, acc.at[dst_all.at[i]], add=True)

    fire(0, rows_a, sem_a)

    @pl.loop(0, HALF - 1)
    def _(j):
        i = 2 * j
        fire(i + 1, rows_b, sem_b)
        wait(rows_a, sem_a)
        scat(i, rows_a)
        fire(i + 2, rows_a, sem_a)
        wait(rows_b, sem_b)
        scat(i + 1, rows_b)

    fire(NITER - 1, rows_b, sem_b)
    wait(rows_a, sem_a)
    scat(NITER - 2, rows_a)
    wait(rows_b, sem_b)
    scat(NITER - 1, rows_b)

    plsc.subcore_barrier()
    pltpu.sync_copy(acc.at[pl.ds(sid * RPT, RPT)],
                    out.at[cid, pl.ds(sid * RPT, RPT)])


def _make_sc_agg(D):
    return pl.kernel(
        _sc_agg_body,
        out_type=jax.ShapeDtypeStruct((NC, NP, D), jnp.float32),
        mesh=_SC_MESH,
        compiler_params=_SC_PARAMS,
        scratch_types=[
            pltpu.VMEM((NITER, CH), jnp.int32),
            pltpu.VMEM((NITER, CH), jnp.int32),
            pltpu.VMEM((CH, D), jnp.float32),
            pltpu.VMEM((CH, D), jnp.float32),
            pltpu.VMEM_SHARED((NP, D), jnp.float32),
            pltpu.VMEM_SHARED((NP, D), jnp.float32),
            pltpu.SemaphoreType.DMA,
            pltpu.SemaphoreType.DMA,
        ],
        name=f"sc_gcn_agg_{D}",
    )


def _sc_deg_body(edst3, ones, zeros, out, dst_all, ones_v, acc, sem):
    cid = lax.axis_index("c")
    sid = lax.axis_index("s")
    wid = cid * NS + sid
    pltpu.sync_copy(zeros.at[pl.ds(sid * RPT, RPT)],
                    acc.at[pl.ds(sid * RPT, RPT)])
    pltpu.sync_copy(edst3.at[wid], dst_all)
    pltpu.sync_copy(ones, ones_v)
    plsc.subcore_barrier()

    @pl.loop(0, NITER)
    def _(i):
        pltpu.async_copy(ones_v, acc.at[dst_all.at[i]], sem, add=True)

    @pl.loop(0, NITER)
    def _(i):
        pltpu.make_async_copy(ones_v, acc.at[dst_all.at[0]], sem).wait()

    plsc.subcore_barrier()
    pltpu.sync_copy(acc.at[pl.ds(sid * RPT, RPT)],
                    out.at[cid, pl.ds(sid * RPT, RPT)])


_sc_deg = pl.kernel(
    _sc_deg_body,
    out_type=jax.ShapeDtypeStruct((NC, NP, DEGW), jnp.float32),
    mesh=_SC_MESH,
    compiler_params=_SC_PARAMS,
    scratch_types=[
        pltpu.VMEM((NITER, CH), jnp.int32),
        pltpu.VMEM((CH, DEGW), jnp.float32),
        pltpu.VMEM_SHARED((NP, DEGW), jnp.float32),
        pltpu.SemaphoreType.DMA,
    ],
    name="sc_gcn_deg",
)

_sc_agg64 = _make_sc_agg(DH)
_sc_agg48 = _make_sc_agg(DCP)


RB = 1024
GRID = NP // RB


def _tc0_body(x_ref, w1_ref, xw_ref):
    xw_ref[...] = jnp.dot(
        x_ref[...], w1_ref[...], preferred_element_type=jnp.float32)


def _tc1_body(xw_ref, degp_ref, dis_ref, xws1_ref):
    deg = degp_ref[0, :, 0:1] + degp_ref[1, :, 0:1] + 1.0
    dis = 1.0 / jnp.sqrt(deg)
    dis_ref[...] = dis
    xws1_ref[...] = xw_ref[...] * dis


def _tc2_body(xws1_ref, p_ref, dis_ref, b1_ref, w2_ref, xws2_ref):
    dis = dis_ref[...]
    h = dis * (p_ref[0] + p_ref[1] + xws1_ref[...]) + b1_ref[...]
    h = jnp.maximum(h, 0.0)
    xws2_ref[...] = jnp.dot(
        h, w2_ref[...], preferred_element_type=jnp.float32) * dis


def _tc3_body(xws2_ref, p_ref, dis_ref, b2_ref, out_ref):
    o = dis_ref[...] * (p_ref[0] + p_ref[1] + xws2_ref[...]) + b2_ref[...]
    mask = lax.broadcasted_iota(jnp.int32, (1, DCP), 1) < DC
    m = jnp.max(jnp.where(mask, o, -jnp.inf), axis=1, keepdims=True)
    s = jnp.sum(jnp.where(mask, jnp.exp(o - m), 0.0), axis=1, keepdims=True)
    out_ref[...] = (o - m - jnp.log(s))[:, :DC]


def _row_spec(d):
    return pl.BlockSpec((RB, d), lambda i: (i, 0))


def _full_spec(shape):
    nd = len(shape)
    return pl.BlockSpec(shape, lambda i: (0,) * nd)


_tc0 = pl.pallas_call(
    _tc0_body,
    grid=(GRID,),
    in_specs=[_row_spec(DIN), _full_spec((DIN, DH))],
    out_specs=_row_spec(DH),
    out_shape=jax.ShapeDtypeStruct((NP, DH), jnp.float32),
)

_tc1 = pl.pallas_call(
    _tc1_body,
    grid=(GRID,),
    in_specs=[_row_spec(DH),
              pl.BlockSpec((NC, RB, DEGW), lambda i: (0, i, 0))],
    out_specs=[_row_spec(1), _row_spec(DH)],
    out_shape=[jax.ShapeDtypeStruct((NP, 1), jnp.float32),
               jax.ShapeDtypeStruct((NP, DH), jnp.float32)],
)

_tc2 = pl.pallas_call(
    _tc2_body,
    grid=(GRID,),
    in_specs=[_row_spec(DH), pl.BlockSpec((NC, RB, DH), lambda i: (0, i, 0)),
              _row_spec(1), _full_spec((1, DH)), _full_spec((DH, DCP))],
    out_specs=_row_spec(DCP),
    out_shape=jax.ShapeDtypeStruct((NP, DCP), jnp.float32),
)

_tc3 = pl.pallas_call(
    _tc3_body,
    grid=(GRID,),
    in_specs=[_row_spec(DCP), pl.BlockSpec((NC, RB, DCP), lambda i: (0, i, 0)),
              _row_spec(1), _full_spec((1, DCP))],
    out_specs=pl.BlockSpec((RB, DC), lambda i: (i, 0)),
    out_shape=jax.ShapeDtypeStruct((N, DC), jnp.float32),
)


def kernel(x, edge_index, W1, b1, W2, b2):
    ei = edge_index.astype(jnp.int32)
    esrc3 = jnp.pad(ei[0], (0, EPAD - E)).reshape(NW, NITER, CH)
    edst3 = jnp.pad(ei[1], (0, EPAD - E),
                    constant_values=NP - 1).reshape(NW, NITER, CH)
    zeros64 = jnp.zeros((NP, DH), jnp.float32)
    zeros48 = jnp.zeros((NP, DCP), jnp.float32)
    zeros_d = jnp.zeros((NP, DEGW), jnp.float32)
    ones_d = jnp.ones((CH, DEGW), jnp.float32)
    w2p = jnp.pad(W2, ((0, 0), (0, DCP - DC)))
    b2p = jnp.pad(b2, (0, DCP - DC)).reshape(1, DCP)
    b1r = b1.reshape(1, DH)

    degp = _sc_deg(edst3, ones_d, zeros_d)
    xw1 = _tc0(x, W1)
    dis, xws1 = _tc1(xw1, degp)
    p1 = _sc_agg64(esrc3, edst3, xws1, zeros64)
    xws2 = _tc2(xws1, p1, dis, b1r, w2p)
    p2 = _sc_agg48(esrc3, edst3, xws2, zeros48)
    return _tc3(xws2, p2, dis, b2p)


# speedup vs baseline: 2.3358x; 1.1213x over previous
import jax
import jax.numpy as jnp
from jax import lax
from jax.experimental import pallas as pl
from jax.experimental.pallas import tpu as pltpu
from jax.experimental.pallas import tpu_sc as plsc

N = 10000
E = 320000
DIN = 128
DH = 64
DC = 40
DEGW = 8
LW = 128

NC, NS = 2, 16
NW = NC * NS
CH = 128
NITER = 80
EPAD = NW * CH * NITER
NP = 10240
RPT = NP // NS
HALF = NITER // 2

_SC_MESH = plsc.VectorSubcoreMesh(core_axis_name="c", subcore_axis_name="s")
_SC_PARAMS = pltpu.CompilerParams(use_tc_tiling_on_sc=False)


def _sc_agg_body(D, esrc3, edst3, table, zeros, out,
                 src_all, dst_all, rows_a, rows_b, tbl, acc, sem_a, sem_b):
    cid = lax.axis_index("c")
    sid = lax.axis_index("s")
    wid = cid * NS + sid
    pltpu.sync_copy(zeros.at[pl.ds(sid * RPT, RPT)],
                    acc.at[pl.ds(sid * RPT, RPT)])
    pltpu.sync_copy(table.at[pl.ds(sid * RPT, RPT), pl.ds(0, D)],
                    tbl.at[pl.ds(sid * RPT, RPT)])
    pltpu.sync_copy(esrc3.at[wid], src_all)
    pltpu.sync_copy(edst3.at[wid], dst_all)
    plsc.subcore_barrier()

    def fire(i, rows, sem):
        pltpu.async_copy(tbl.at[src_all.at[i]], rows, sem)

    def wait(rows, sem):
        pltpu.make_async_copy(tbl.at[src_all.at[0]], rows, sem).wait()

    def scat(i, rows):
        pltpu.sync_copy(rows, acc.at[dst_all.at[i]], add=True)

    fire(0, rows_a, sem_a)

    @pl.loop(0, HALF - 1)
    def _(j):
        i = 2 * j
        fire(i + 1, rows_b, sem_b)
        wait(rows_a, sem_a)
        scat(i, rows_a)
        fire(i + 2, rows_a, sem_a)
        wait(rows_b, sem_b)
        scat(i + 1, rows_b)

    fire(NITER - 1, rows_b, sem_b)
    wait(rows_a, sem_a)
    scat(NITER - 2, rows_a)
    wait(rows_b, sem_b)
    scat(NITER - 1, rows_b)

    plsc.subcore_barrier()
    pltpu.sync_copy(acc.at[pl.ds(sid * RPT, RPT)],
                    out.at[pl.ds(sid * RPT, RPT), pl.ds(cid * DH, D)])


def _make_sc_agg(D):
    import functools
    return pl.kernel(
        functools.partial(_sc_agg_body, D),
        out_type=jax.ShapeDtypeStruct((NP, LW), jnp.float32),
        mesh=_SC_MESH,
        compiler_params=_SC_PARAMS,
        scratch_types=[
            pltpu.VMEM((NITER, CH), jnp.int32),
            pltpu.VMEM((NITER, CH), jnp.int32),
            pltpu.VMEM((CH, D), jnp.float32),
            pltpu.VMEM((CH, D), jnp.float32),
            pltpu.VMEM_SHARED((NP, D), jnp.float32),
            pltpu.VMEM_SHARED((NP, D), jnp.float32),
            pltpu.SemaphoreType.DMA,
            pltpu.SemaphoreType.DMA,
        ],
        name=f"sc_gcn_agg_{D}",
    )


def _sc_deg_body(edst3, ones, zeros, out, dst_all, ones_v, acc, sem):
    cid = lax.axis_index("c")
    sid = lax.axis_index("s")
    wid = cid * NS + sid
    pltpu.sync_copy(zeros.at[pl.ds(sid * RPT, RPT)],
                    acc.at[pl.ds(sid * RPT, RPT)])
    pltpu.sync_copy(edst3.at[wid], dst_all)
    pltpu.sync_copy(ones, ones_v)
    plsc.subcore_barrier()

    @pl.loop(0, NITER)
    def _(i):
        pltpu.async_copy(ones_v, acc.at[dst_all.at[i]], sem, add=True)

    @pl.loop(0, NITER)
    def _(i):
        pltpu.make_async_copy(ones_v, acc.at[dst_all.at[0]], sem).wait()

    plsc.subcore_barrier()
    pltpu.sync_copy(acc.at[pl.ds(sid * RPT, RPT)],
                    out.at[cid, pl.ds(sid * RPT, RPT)])


_sc_deg = pl.kernel(
    _sc_deg_body,
    out_type=jax.ShapeDtypeStruct((NC, NP, DEGW), jnp.float32),
    mesh=_SC_MESH,
    compiler_params=_SC_PARAMS,
    scratch_types=[
        pltpu.VMEM((NITER, CH), jnp.int32),
        pltpu.VMEM((CH, DEGW), jnp.float32),
        pltpu.VMEM_SHARED((NP, DEGW), jnp.float32),
        pltpu.SemaphoreType.DMA,
    ],
    name="sc_gcn_deg",
)

_sc_agg64 = _make_sc_agg(DH)
_sc_agg40 = _make_sc_agg(DC)


RB = 1024
GRID = NP // RB


def _tc0_body(x_ref, w1_ref, xw_ref):
    xw_ref[...] = jnp.dot(
        x_ref[...], w1_ref[...], preferred_element_type=jnp.float32)


def _tc1_body(xw_ref, degp_ref, dis_ref, xws1_ref):
    deg = degp_ref[0, :, 0:1] + degp_ref[1, :, 0:1] + 1.0
    dis = 1.0 / jnp.sqrt(deg)
    dis_ref[...] = dis
    xws1_ref[...] = jnp.concatenate(
        [xw_ref[...] * dis, jnp.zeros((RB, LW - DH), jnp.float32)], axis=1)


def _tc2_body(xws1_ref, p_ref, dis_ref, b1_ref, w2_ref, xws2_ref):
    dis = dis_ref[...]
    p = p_ref[...]
    h = dis * (p[:, :DH] + p[:, DH:] + xws1_ref[:, :DH]) + b1_ref[...]
    h = jnp.maximum(h, 0.0)
    xws2_ref[...] = jnp.dot(
        h, w2_ref[...], preferred_element_type=jnp.float32) * dis


def _tc3_body(xws2_ref, p_ref, dis_ref, b2_ref, out_ref):
    p = p_ref[...]
    o = dis_ref[...] * (p[:, :DC] + p[:, DH:DH + DC] + xws2_ref[:, :DC])
    o = o + b2_ref[...]
    m = jnp.max(o, axis=1, keepdims=True)
    s = jnp.sum(jnp.exp(o - m), axis=1, keepdims=True)
    out_ref[...] = o - m - jnp.log(s)


def _row_spec(d):
    return pl.BlockSpec((RB, d), lambda i: (i, 0))


def _full_spec(shape):
    nd = len(shape)
    return pl.BlockSpec(shape, lambda i: (0,) * nd)


_tc0 = pl.pallas_call(
    _tc0_body,
    grid=(GRID,),
    in_specs=[_row_spec(DIN), _full_spec((DIN, DH))],
    out_specs=_row_spec(DH),
    out_shape=jax.ShapeDtypeStruct((NP, DH), jnp.float32),
)

_tc1 = pl.pallas_call(
    _tc1_body,
    grid=(GRID,),
    in_specs=[_row_spec(DH),
              pl.BlockSpec((NC, RB, DEGW), lambda i: (0, i, 0))],
    out_specs=[_row_spec(1), _row_spec(LW)],
    out_shape=[jax.ShapeDtypeStruct((NP, 1), jnp.float32),
               jax.ShapeDtypeStruct((NP, LW), jnp.float32)],
)

_tc2 = pl.pallas_call(
    _tc2_body,
    grid=(GRID,),
    in_specs=[_row_spec(LW), _row_spec(LW),
              _row_spec(1), _full_spec((1, DH)), _full_spec((DH, LW))],
    out_specs=_row_spec(LW),
    out_shape=jax.ShapeDtypeStruct((NP, LW), jnp.float32),
)

_tc3 = pl.pallas_call(
    _tc3_body,
    grid=(GRID,),
    in_specs=[_row_spec(LW), _row_spec(LW),
              _row_spec(1), _full_spec((1, DC))],
    out_specs=pl.BlockSpec((RB, DC), lambda i: (i, 0)),
    out_shape=jax.ShapeDtypeStruct((N, DC), jnp.float32),
)


def kernel(x, edge_index, W1, b1, W2, b2):
    ei = edge_index.astype(jnp.int32)
    esrc3 = jnp.pad(ei[0], (0, EPAD - E)).reshape(NW, NITER, CH)
    edst3 = jnp.pad(ei[1], (0, EPAD - E),
                    constant_values=NP - 1).reshape(NW, NITER, CH)
    zeros64 = jnp.zeros((NP, DH), jnp.float32)
    zeros40 = jnp.zeros((NP, DC), jnp.float32)
    zeros_d = jnp.zeros((NP, DEGW), jnp.float32)
    ones_d = jnp.ones((CH, DEGW), jnp.float32)
    w2p = jnp.pad(W2, ((0, 0), (0, LW - DC)))
    b2r = b2.reshape(1, DC)
    b1r = b1.reshape(1, DH)

    degp = _sc_deg(edst3, ones_d, zeros_d)
    xw1 = _tc0(x, W1)
    dis, xws1 = _tc1(xw1, degp)
    p1 = _sc_agg64(esrc3, edst3, xws1, zeros64)
    xws2 = _tc2(xws1, p1, dis, b1r, w2p)
    p2 = _sc_agg40(esrc3, edst3, xws2, zeros40)
    return _tc3(xws2, p2, dis, b2r)


# speedup vs baseline: 2.4513x; 1.0495x over previous
import jax
import jax.numpy as jnp
from jax import lax
from jax.experimental import pallas as pl
from jax.experimental.pallas import tpu as pltpu
from jax.experimental.pallas import tpu_sc as plsc

N = 10000
E = 320000
DIN = 128
DH = 64
DC = 40
DEGW = 8
LW = 128

NC, NS = 2, 16
NW = NC * NS
CH = 125
NITER = 80
NP = 10240
RPT = NP // NS
HALF = NITER // 2

_SC_MESH = plsc.VectorSubcoreMesh(core_axis_name="c", subcore_axis_name="s")
_SC_PARAMS = pltpu.CompilerParams(use_tc_tiling_on_sc=False)


def _sc_agg_body(D, esrc3, edst3, table, zeros, out,
                 src_all, dst_all, rows_a, rows_b, tbl, acc, sem_a, sem_b):
    cid = lax.axis_index("c")
    sid = lax.axis_index("s")
    wid = cid * NS + sid
    pltpu.sync_copy(zeros.at[pl.ds(sid * RPT, RPT)],
                    acc.at[pl.ds(sid * RPT, RPT)])
    pltpu.sync_copy(table.at[pl.ds(sid * RPT, RPT), pl.ds(0, D)],
                    tbl.at[pl.ds(sid * RPT, RPT)])
    pltpu.sync_copy(esrc3.at[wid], src_all)
    pltpu.sync_copy(edst3.at[wid], dst_all)
    plsc.subcore_barrier()

    def fire(i, rows, sem):
        pltpu.async_copy(tbl.at[src_all.at[i]], rows, sem)

    def wait(rows, sem):
        pltpu.make_async_copy(tbl.at[src_all.at[0]], rows, sem).wait()

    def scat(i, rows):
        pltpu.sync_copy(rows, acc.at[dst_all.at[i]], add=True)

    fire(0, rows_a, sem_a)

    @pl.loop(0, HALF - 1)
    def _(j):
        i = 2 * j
        fire(i + 1, rows_b, sem_b)
        wait(rows_a, sem_a)
        scat(i, rows_a)
        fire(i + 2, rows_a, sem_a)
        wait(rows_b, sem_b)
        scat(i + 1, rows_b)

    fire(NITER - 1, rows_b, sem_b)
    wait(rows_a, sem_a)
    scat(NITER - 2, rows_a)
    wait(rows_b, sem_b)
    scat(NITER - 1, rows_b)

    plsc.subcore_barrier()
    pltpu.sync_copy(acc.at[pl.ds(sid * RPT, RPT)],
                    out.at[pl.ds(sid * RPT, RPT), pl.ds(cid * DH, D)])


def _make_sc_agg(D):
    import functools
    return pl.kernel(
        functools.partial(_sc_agg_body, D),
        out_type=jax.ShapeDtypeStruct((NP, LW), jnp.float32),
        mesh=_SC_MESH,
        compiler_params=_SC_PARAMS,
        scratch_types=[
            pltpu.VMEM((NITER, CH), jnp.int32),
            pltpu.VMEM((NITER, CH), jnp.int32),
            pltpu.VMEM((CH, D), jnp.float32),
            pltpu.VMEM((CH, D), jnp.float32),
            pltpu.VMEM_SHARED((NP, D), jnp.float32),
            pltpu.VMEM_SHARED((NP, D), jnp.float32),
            pltpu.SemaphoreType.DMA,
            pltpu.SemaphoreType.DMA,
        ],
        name=f"sc_gcn_agg_{D}",
    )


def _sc_deg_body(edst3, ones, zeros, out, dst_all, ones_v, acc, sem):
    cid = lax.axis_index("c")
    sid = lax.axis_index("s")
    wid = cid * NS + sid
    pltpu.sync_copy(zeros.at[pl.ds(sid * RPT, RPT)],
                    acc.at[pl.ds(sid * RPT, RPT)])
    pltpu.sync_copy(edst3.at[wid], dst_all)
    pltpu.sync_copy(ones, ones_v)
    plsc.subcore_barrier()

    @pl.loop(0, NITER)
    def _(i):
        pltpu.async_copy(ones_v, acc.at[dst_all.at[i]], sem, add=True)

    @pl.loop(0, NITER)
    def _(i):
        pltpu.make_async_copy(ones_v, acc.at[dst_all.at[0]], sem).wait()

    plsc.subcore_barrier()
    pltpu.sync_copy(acc.at[pl.ds(sid * RPT, RPT)],
                    out.at[pl.ds(sid * RPT, RPT), pl.ds(cid * DH, DEGW)])


_sc_deg = pl.kernel(
    _sc_deg_body,
    out_type=jax.ShapeDtypeStruct((NP, LW), jnp.float32),
    mesh=_SC_MESH,
    compiler_params=_SC_PARAMS,
    scratch_types=[
        pltpu.VMEM((NITER, CH), jnp.int32),
        pltpu.VMEM((CH, DEGW), jnp.float32),
        pltpu.VMEM_SHARED((NP, DEGW), jnp.float32),
        pltpu.SemaphoreType.DMA,
    ],
    name="sc_gcn_deg",
)

_sc_agg64 = _make_sc_agg(DH)
_sc_agg40 = _make_sc_agg(DC)


RB = 1024
GRID = NP // RB


def _tc0_body(x_ref, w1_ref, xw_ref):
    xw_ref[...] = jnp.dot(
        x_ref[...], w1_ref[...], preferred_element_type=jnp.float32)


def _tc1_body(xw_ref, degp_ref, dis_ref, xws1_ref):
    deg = degp_ref[:, 0:1] + degp_ref[:, DH:DH + 1] + 1.0
    dis = 1.0 / jnp.sqrt(deg)
    dis_ref[...] = dis
    xws1_ref[...] = jnp.concatenate(
        [xw_ref[...] * dis, jnp.zeros((RB, LW - DH), jnp.float32)], axis=1)


def _tc2_body(xws1_ref, p_ref, dis_ref, b1_ref, w2_ref, xws2_ref):
    dis = dis_ref[...]
    p = p_ref[...]
    h = dis * (p[:, :DH] + p[:, DH:] + xws1_ref[:, :DH]) + b1_ref[...]
    h = jnp.maximum(h, 0.0)
    xws2_ref[...] = jnp.dot(
        h, w2_ref[...], preferred_element_type=jnp.float32) * dis


def _tc3_body(xws2_ref, p_ref, dis_ref, b2_ref, out_ref):
    p = p_ref[...]
    o = dis_ref[...] * (p[:, :DC] + p[:, DH:DH + DC] + xws2_ref[:, :DC])
    o = o + b2_ref[...]
    m = jnp.max(o, axis=1, keepdims=True)
    s = jnp.sum(jnp.exp(o - m), axis=1, keepdims=True)
    out_ref[...] = o - m - jnp.log(s)


def _row_spec(d):
    return pl.BlockSpec((RB, d), lambda i: (i, 0))


def _full_spec(shape):
    nd = len(shape)
    return pl.BlockSpec(shape, lambda i: (0,) * nd)


_tc0 = pl.pallas_call(
    _tc0_body,
    grid=(GRID,),
    in_specs=[_row_spec(DIN), _full_spec((DIN, DH))],
    out_specs=_row_spec(DH),
    out_shape=jax.ShapeDtypeStruct((NP, DH), jnp.float32),
)

_tc1 = pl.pallas_call(
    _tc1_body,
    grid=(GRID,),
    in_specs=[_row_spec(DH), _row_spec(LW)],
    out_specs=[_row_spec(1), _row_spec(LW)],
    out_shape=[jax.ShapeDtypeStruct((NP, 1), jnp.float32),
               jax.ShapeDtypeStruct((NP, LW), jnp.float32)],
)

_tc2 = pl.pallas_call(
    _tc2_body,
    grid=(GRID,),
    in_specs=[_row_spec(LW), _row_spec(LW),
              _row_spec(1), _full_spec((1, DH)), _full_spec((DH, LW))],
    out_specs=_row_spec(LW),
    out_shape=jax.ShapeDtypeStruct((NP, LW), jnp.float32),
)

_tc3 = pl.pallas_call(
    _tc3_body,
    grid=(GRID,),
    in_specs=[_row_spec(LW), _row_spec(LW),
              _row_spec(1), _full_spec((1, DC))],
    out_specs=pl.BlockSpec((RB, DC), lambda i: (i, 0)),
    out_shape=jax.ShapeDtypeStruct((N, DC), jnp.float32),
)


def kernel(x, edge_index, W1, b1, W2, b2):
    ei = edge_index.astype(jnp.int32)
    esrc3 = ei[0].reshape(NW, NITER, CH)
    edst3 = ei[1].reshape(NW, NITER, CH)
    zeros64 = jnp.zeros((NP, DH), jnp.float32)
    zeros40 = jnp.zeros((NP, DC), jnp.float32)
    zeros_d = jnp.zeros((NP, DEGW), jnp.float32)
    ones_d = jnp.ones((CH, DEGW), jnp.float32)
    w2p = jnp.pad(W2, ((0, 0), (0, LW - DC)))
    b2r = b2.reshape(1, DC)
    b1r = b1.reshape(1, DH)

    degp = _sc_deg(edst3, ones_d, zeros_d)
    xw1 = _tc0(x, W1)
    dis, xws1 = _tc1(xw1, degp)
    p1 = _sc_agg64(esrc3, edst3, xws1, zeros64)
    xws2 = _tc2(xws1, p1, dis, b1r, w2p)
    p2 = _sc_agg40(esrc3, edst3, xws2, zeros40)
    return _tc3(xws2, p2, dis, b2r)
